# baseline (speedup 1.0000x reference)
import jax
import jax.numpy as jnp
from jax import lax
from jax.experimental import pallas as pl
from jax.experimental.pallas import tpu as pltpu
from jax.experimental.pallas import tpu_sc as plsc

N_NODES = 10000
N_EDGES = 320000
IN_CH = 128
OUT_CH = 128

NC = 2
NS = 16
L = 16

NPAD = 10240
ROWS_PER_TILE = NPAD // NS
CHUNK = 64
QUAD = 4
NW = NC * NS
NCHUNKS = N_EDGES // CHUNK
SG = 4
NSG = 313
DCHUNK = 128
DEG_CPW = 80
DEG_CHUNKS = NW * DEG_CPW

_sc_mesh = plsc.VectorSubcoreMesh(
    core_axis_name="c", subcore_axis_name="s", num_cores=NC, num_subcores=NS
)


def _deg_body(dst_hbm, ew_hbm, deg_hbm, dstv, ewv, degsh, zb, sem):
    c = lax.axis_index("c")
    s = lax.axis_index("s")
    wid = c * NS + s
    base = wid * DEG_CPW
    pltpu.sync_copy(dst_hbm.at[pl.ds(base, DEG_CPW)], dstv)
    pltpu.sync_copy(ew_hbm.at[pl.ds(base, DEG_CPW)], ewv)

    def zloop(i, carry):
        zb[pl.ds(i * L, L)] = jnp.zeros((L,), jnp.float32)
        return carry

    lax.fori_loop(0, ROWS_PER_TILE // L, zloop, 0)
    pltpu.sync_copy(zb, degsh.at[pl.ds(s * ROWS_PER_TILE, ROWS_PER_TILE)])
    plsc.subcore_barrier()

    def sloop(j, carry):
        pltpu.async_copy(ewv.at[j], degsh.at[dstv.at[j]], sem, add=True)
        return carry

    lax.fori_loop(0, DEG_CPW, sloop, 0)

    def dloop(j, carry):
        pltpu.make_async_copy(ewv.at[j], degsh.at[dstv.at[j]], sem).wait()
        return carry

    lax.fori_loop(0, DEG_CPW, dloop, 0)
    plsc.subcore_barrier()
    pltpu.sync_copy(degsh.at[pl.ds(s * ROWS_PER_TILE, ROWS_PER_TILE)], zb)
    pltpu.sync_copy(zb, deg_hbm.at[pl.ds(c * NPAD + s * ROWS_PER_TILE, ROWS_PER_TILE)])


def _deg(dst2, ew2):
    return pl.kernel(
        _deg_body,
        out_type=jax.ShapeDtypeStruct((NC * NPAD,), jnp.float32),
        mesh=_sc_mesh,
        scratch_types=[
            pltpu.VMEM((DEG_CPW, DCHUNK), jnp.int32),
            pltpu.VMEM((DEG_CPW, DCHUNK), jnp.float32),
            pltpu.VMEM_SHARED((NPAD,), jnp.float32),
            pltpu.VMEM((ROWS_PER_TILE,), jnp.float32),
            pltpu.SemaphoreType.DMA,
        ],
    )(dst2, ew2)


def _hs_body(deg_ref, x_ref, w_ref, hs_ref, dis_ref):
    d = deg_ref[:, 0] + deg_ref[:, 1] + 1.0
    dis = lax.rsqrt(d)
    h = jnp.dot(x_ref[...], w_ref[...], preferred_element_type=jnp.float32)
    hs_ref[...] = h * dis[:, None]
    dis_ref[...] = dis[:, None]


def _hs(deg2, xp, W):
    blk = 512
    return pl.pallas_call(
        _hs_body,
        grid=(NPAD // blk,),
        in_specs=[
            pl.BlockSpec((blk, NC), lambda i: (i, 0)),
            pl.BlockSpec((blk, IN_CH), lambda i: (i, 0)),
            pl.BlockSpec((IN_CH, OUT_CH), lambda i: (0, 0)),
        ],
        out_specs=[
            pl.BlockSpec((blk, OUT_CH), lambda i: (i, 0)),
            pl.BlockSpec((blk, 1), lambda i: (i, 0)),
        ],
        out_shape=[
            jax.ShapeDtypeStruct((NPAD, OUT_CH), jnp.float32),
            jax.ShapeDtypeStruct((NPAD, 1), jnp.float32),
        ],
    )(deg2, xp, W)


def _row_scale(rowb, ewv, j):
    def scale(rg, carry):
        n16 = ewv[j, pl.ds(rg * L, L)]
        for i in range(L):
            nb = n16.at[jnp.full((L,), i, jnp.int32)].get(
                mode="promise_in_bounds")
            r = rg * L + i
            for q in range(OUT_CH // L):
                sl = pl.ds(q * L, L)
                rowb[r, sl] = rowb[r, sl] * nb
        return carry

    lax.fori_loop(0, CHUNK // L, scale, 0)


def _msg_body(src_hbm, dst_hbm, ew_hbm, hs_hbm, out_hbm,
              srcv, dstv, ewv, row0, row1, row2, row3, accsh,
              isem, gs0, gs1, gs2, gs3, ss0, ss1, ss2, ss3):
    c = lax.axis_index("c")
    s = lax.axis_index("s")
    wid = c * NS + s
    rows = [row0, row1, row2, row3]
    gsems = [gs0, gs1, gs2, gs3]
    ssems = [ss0, ss1, ss2, ss3]
    gmax = (NSG - 1 - wid) // NW + 1

    def zr(r, carry):
        for q in range(OUT_CH // L):
            row0[r, pl.ds(q * L, L)] = jnp.zeros((L,), jnp.float32)
        return carry

    lax.fori_loop(0, CHUNK, zr, 0)
    for k in range(ROWS_PER_TILE // CHUNK):
        pltpu.sync_copy(row0, accsh.at[pl.ds(s * ROWS_PER_TILE + k * CHUNK, CHUNK)])
    plsc.subcore_barrier()

    def sgbody(t, carry):
        cb = SG * QUAD * (wid + NW * t)
        n = SG * QUAD
        pltpu.async_copy(src_hbm.at[pl.ds(cb, n)], srcv, isem)
        pltpu.async_copy(dst_hbm.at[pl.ds(cb, n)], dstv, isem)
        pltpu.async_copy(ew_hbm.at[pl.ds(cb, n)], ewv, isem)
        pltpu.make_async_copy(src_hbm.at[pl.ds(cb, n)], srcv, isem).wait()
        pltpu.make_async_copy(dst_hbm.at[pl.ds(cb, n)], dstv, isem).wait()
        pltpu.make_async_copy(ew_hbm.at[pl.ds(cb, n)], ewv, isem).wait()
        for q in range(SG):
            for i in range(QUAD):
                j = q * QUAD + i
                pltpu.async_copy(hs_hbm.at[srcv.at[j]], rows[i], gsems[i])
            for i in range(QUAD):
                j = q * QUAD + i
                pltpu.make_async_copy(
                    hs_hbm.at[srcv.at[j]], rows[i], gsems[i]).wait()
                _row_scale(rows[i], ewv, j)
                pltpu.async_copy(rows[i], accsh.at[dstv.at[j]], ssems[i], add=True)
            for i in range(QUAD):
                j = q * QUAD + i
                pltpu.make_async_copy(
                    rows[i], accsh.at[dstv.at[j]], ssems[i]).wait()
        return carry

    lax.fori_loop(0, gmax, sgbody, 0)
    plsc.subcore_barrier()
    for k in range(ROWS_PER_TILE // CHUNK):
        off = s * ROWS_PER_TILE + k * CHUNK
        pltpu.sync_copy(accsh.at[pl.ds(off, CHUNK)], row0)
        pltpu.sync_copy(row0, out_hbm.at[c, pl.ds(off, CHUNK)])


def _msg(src2, dst2, ew2, hs):
    return pl.kernel(
        _msg_body,
        out_type=jax.ShapeDtypeStruct((NC, NPAD, OUT_CH), jnp.float32),
        mesh=_sc_mesh,
        scratch_types=[
            pltpu.VMEM((SG * QUAD, CHUNK), jnp.int32),
            pltpu.VMEM((SG * QUAD, CHUNK), jnp.int32),
            pltpu.VMEM((SG * QUAD, CHUNK), jnp.float32),
            pltpu.VMEM((CHUNK, OUT_CH), jnp.float32),
            pltpu.VMEM((CHUNK, OUT_CH), jnp.float32),
            pltpu.VMEM((CHUNK, OUT_CH), jnp.float32),
            pltpu.VMEM((CHUNK, OUT_CH), jnp.float32),
            pltpu.VMEM_SHARED((NPAD, OUT_CH), jnp.float32),
            pltpu.SemaphoreType.DMA,
            pltpu.SemaphoreType.DMA,
            pltpu.SemaphoreType.DMA,
            pltpu.SemaphoreType.DMA,
            pltpu.SemaphoreType.DMA,
            pltpu.SemaphoreType.DMA,
            pltpu.SemaphoreType.DMA,
            pltpu.SemaphoreType.DMA,
            pltpu.SemaphoreType.DMA,
        ],
    )(src2, dst2, ew2, hs)


def _comb_body(p0_ref, p1_ref, hs_ref, dis_ref, b_ref, o_ref):
    o_ref[...] = (p0_ref[...] + p1_ref[...] + hs_ref[...]) * dis_ref[...] + b_ref[...]


def _combine(p0, p1, hs, dis2, b2):
    blk = 512
    return pl.pallas_call(
        _comb_body,
        grid=(NPAD // blk,),
        in_specs=[
            pl.BlockSpec((blk, OUT_CH), lambda i: (i, 0)),
            pl.BlockSpec((blk, OUT_CH), lambda i: (i, 0)),
            pl.BlockSpec((blk, OUT_CH), lambda i: (i, 0)),
            pl.BlockSpec((blk, 1), lambda i: (i, 0)),
            pl.BlockSpec((1, OUT_CH), lambda i: (0, 0)),
        ],
        out_specs=pl.BlockSpec((blk, OUT_CH), lambda i: (i, 0)),
        out_shape=jax.ShapeDtypeStruct((NPAD, OUT_CH), jnp.float32),
    )(p0, p1, hs, dis2, b2)


def kernel(x, edge_index, edge_weight, W, b):
    nfill = DEG_CHUNKS * DCHUNK - N_EDGES
    src_f = jnp.concatenate(
        [edge_index[0].astype(jnp.int32), jnp.zeros((nfill,), jnp.int32)])
    dst_f = jnp.concatenate(
        [edge_index[1].astype(jnp.int32), jnp.zeros((nfill,), jnp.int32)])
    ew_f = jnp.concatenate(
        [edge_weight.astype(jnp.float32), jnp.zeros((nfill,), jnp.float32)])
    xp = jnp.pad(x.astype(jnp.float32), ((0, NPAD - N_NODES), (0, 0)))

    deg = _deg(dst_f.reshape(DEG_CHUNKS, DCHUNK), ew_f.reshape(DEG_CHUNKS, DCHUNK))
    hs, dis2 = _hs(deg.reshape(NC, NPAD).T, xp, W.astype(jnp.float32))
    parts = _msg(src_f.reshape(-1, CHUNK), dst_f.reshape(-1, CHUNK),
                 ew_f.reshape(-1, CHUNK), hs)
    out = _combine(parts[0], parts[1], hs, dis2,
                   b.astype(jnp.float32).reshape(1, OUT_CH))
    return out[:N_NODES]


# speedup vs baseline: 1.0621x; 1.0621x over previous
import jax
import jax.numpy as jnp
from jax import lax
from jax.experimental import pallas as pl
from jax.experimental.pallas import tpu as pltpu
from jax.experimental.pallas import tpu_sc as plsc

N_NODES = 10000
N_EDGES = 320000
IN_CH = 128
OUT_CH = 128

NC = 2
NS = 16
L = 16

NPAD = 10240
ROWS_PER_TILE = NPAD // NS
CHUNK = 64
QUAD = 4
NW = NC * NS
NCHUNKS = N_EDGES // CHUNK
NQUADS = NCHUNKS // QUAD
DCHUNK = 128
DEG_CPW = 80
DEG_CHUNKS = NW * DEG_CPW

_sc_mesh = plsc.VectorSubcoreMesh(
    core_axis_name="c", subcore_axis_name="s", num_cores=NC, num_subcores=NS
)


def _deg_body(dst_hbm, ew_hbm, deg_hbm, dstv, ewv, degsh, zb, sem):
    c = lax.axis_index("c")
    s = lax.axis_index("s")
    wid = c * NS + s
    base = wid * DEG_CPW
    pltpu.sync_copy(dst_hbm.at[pl.ds(base, DEG_CPW)], dstv)
    pltpu.sync_copy(ew_hbm.at[pl.ds(base, DEG_CPW)], ewv)

    def zloop(i, carry):
        zb[pl.ds(i * L, L)] = jnp.zeros((L,), jnp.float32)
        return carry

    lax.fori_loop(0, ROWS_PER_TILE // L, zloop, 0)
    pltpu.sync_copy(zb, degsh.at[pl.ds(s * ROWS_PER_TILE, ROWS_PER_TILE)])
    plsc.subcore_barrier()

    def sloop(j, carry):
        pltpu.async_copy(ewv.at[j], degsh.at[dstv.at[j]], sem, add=True)
        return carry

    lax.fori_loop(0, DEG_CPW, sloop, 0)

    def dloop(j, carry):
        pltpu.make_async_copy(ewv.at[j], degsh.at[dstv.at[j]], sem).wait()
        return carry

    lax.fori_loop(0, DEG_CPW, dloop, 0)
    plsc.subcore_barrier()
    pltpu.sync_copy(degsh.at[pl.ds(s * ROWS_PER_TILE, ROWS_PER_TILE)], zb)
    pltpu.sync_copy(zb, deg_hbm.at[pl.ds(c * NPAD + s * ROWS_PER_TILE, ROWS_PER_TILE)])


def _deg(dst2, ew2):
    return pl.kernel(
        _deg_body,
        out_type=jax.ShapeDtypeStruct((NC * NPAD,), jnp.float32),
        mesh=_sc_mesh,
        scratch_types=[
            pltpu.VMEM((DEG_CPW, DCHUNK), jnp.int32),
            pltpu.VMEM((DEG_CPW, DCHUNK), jnp.float32),
            pltpu.VMEM_SHARED((NPAD,), jnp.float32),
            pltpu.VMEM((ROWS_PER_TILE,), jnp.float32),
            pltpu.SemaphoreType.DMA,
        ],
    )(dst2, ew2)


def _hs_body(deg_ref, x_ref, w_ref, hs_ref, dis_ref):
    d = deg_ref[:, 0] + deg_ref[:, 1] + 1.0
    dis = lax.rsqrt(d)
    h = jnp.dot(x_ref[...], w_ref[...], preferred_element_type=jnp.float32)
    hs_ref[...] = h * dis[:, None]
    dis_ref[...] = dis[:, None]


def _hs(deg2, xp, W):
    blk = 512
    return pl.pallas_call(
        _hs_body,
        grid=(NPAD // blk,),
        in_specs=[
            pl.BlockSpec((blk, NC), lambda i: (i, 0)),
            pl.BlockSpec((blk, IN_CH), lambda i: (i, 0)),
            pl.BlockSpec((IN_CH, OUT_CH), lambda i: (0, 0)),
        ],
        out_specs=[
            pl.BlockSpec((blk, OUT_CH), lambda i: (i, 0)),
            pl.BlockSpec((blk, 1), lambda i: (i, 0)),
        ],
        out_shape=[
            jax.ShapeDtypeStruct((NPAD, OUT_CH), jnp.float32),
            jax.ShapeDtypeStruct((NPAD, 1), jnp.float32),
        ],
    )(deg2, xp, W)


def _row_scale(rowb, ewv, j):
    def scale(rg, carry):
        n16 = ewv[j, pl.ds(rg * L, L)]
        for i in range(L):
            nb = n16.at[jnp.full((L,), i, jnp.int32)].get(
                mode="promise_in_bounds")
            r = rg * L + i
            for q in range(OUT_CH // L):
                sl = pl.ds(q * L, L)
                rowb[r, sl] = rowb[r, sl] * nb
        return carry

    lax.fori_loop(0, CHUNK // L, scale, 0)


def _msg_body(src_hbm, dst_hbm, ew_hbm, hs_hbm, out_hbm,
              srcv, dstv, ewv, row0, row1, row2, row3, accsh,
              isem, gs0, gs1, gs2, gs3, ss0, ss1, ss2, ss3):
    c = lax.axis_index("c")
    s = lax.axis_index("s")
    wid = c * NS + s
    rows = [row0, row1, row2, row3]
    gsems = [gs0, gs1, gs2, gs3]
    ssems = [ss0, ss1, ss2, ss3]
    qmax = (NQUADS - 1 - wid) // NW + 1

    def zr(r, carry):
        for q in range(OUT_CH // L):
            row0[r, pl.ds(q * L, L)] = jnp.zeros((L,), jnp.float32)
        return carry

    lax.fori_loop(0, CHUNK, zr, 0)
    for k in range(ROWS_PER_TILE // CHUNK):
        pltpu.sync_copy(row0, accsh.at[pl.ds(s * ROWS_PER_TILE + k * CHUNK, CHUNK)])
    plsc.subcore_barrier()

    def qbody(t, carry):
        cb = QUAD * (wid + NW * t)
        pltpu.async_copy(src_hbm.at[pl.ds(cb, QUAD)], srcv, isem)
        pltpu.async_copy(dst_hbm.at[pl.ds(cb, QUAD)], dstv, isem)
        pltpu.async_copy(ew_hbm.at[pl.ds(cb, QUAD)], ewv, isem)
        pltpu.make_async_copy(src_hbm.at[pl.ds(cb, QUAD)], srcv, isem).wait()
        pltpu.make_async_copy(dst_hbm.at[pl.ds(cb, QUAD)], dstv, isem).wait()
        pltpu.make_async_copy(ew_hbm.at[pl.ds(cb, QUAD)], ewv, isem).wait()
        for i in range(QUAD):
            pltpu.async_copy(hs_hbm.at[srcv.at[i]], rows[i], gsems[i])
        for i in range(QUAD):
            pltpu.make_async_copy(hs_hbm.at[srcv.at[i]], rows[i], gsems[i]).wait()
            _row_scale(rows[i], ewv, i)
            pltpu.async_copy(rows[i], accsh.at[dstv.at[i]], ssems[i], add=True)
        for i in range(QUAD):
            pltpu.make_async_copy(rows[i], accsh.at[dstv.at[i]], ssems[i]).wait()
        return carry

    lax.fori_loop(0, qmax, qbody, 0)
    plsc.subcore_barrier()
    for k in range(ROWS_PER_TILE // CHUNK):
        off = s * ROWS_PER_TILE + k * CHUNK
        pltpu.sync_copy(accsh.at[pl.ds(off, CHUNK)], row0)
        pltpu.sync_copy(row0, out_hbm.at[c, pl.ds(off, CHUNK)])


def _msg(src2, dst2, ew2, hs):
    return pl.kernel(
        _msg_body,
        out_type=jax.ShapeDtypeStruct((NC, NPAD, OUT_CH), jnp.float32),
        mesh=_sc_mesh,
        scratch_types=[
            pltpu.VMEM((QUAD, CHUNK), jnp.int32),
            pltpu.VMEM((QUAD, CHUNK), jnp.int32),
            pltpu.VMEM((QUAD, CHUNK), jnp.float32),
            pltpu.VMEM((CHUNK, OUT_CH), jnp.float32),
            pltpu.VMEM((CHUNK, OUT_CH), jnp.float32),
            pltpu.VMEM((CHUNK, OUT_CH), jnp.float32),
            pltpu.VMEM((CHUNK, OUT_CH), jnp.float32),
            pltpu.VMEM_SHARED((NPAD, OUT_CH), jnp.float32),
            pltpu.SemaphoreType.DMA,
            pltpu.SemaphoreType.DMA,
            pltpu.SemaphoreType.DMA,
            pltpu.SemaphoreType.DMA,
            pltpu.SemaphoreType.DMA,
            pltpu.SemaphoreType.DMA,
            pltpu.SemaphoreType.DMA,
            pltpu.SemaphoreType.DMA,
            pltpu.SemaphoreType.DMA,
        ],
    )(src2, dst2, ew2, hs)


def _comb_body(p0_ref, p1_ref, hs_ref, dis_ref, b_ref, o_ref):
    o_ref[...] = (p0_ref[...] + p1_ref[...] + hs_ref[...]) * dis_ref[...] + b_ref[...]


def _combine(p0, p1, hs, dis2, b2):
    blk = 1000
    return pl.pallas_call(
        _comb_body,
        grid=(N_NODES // blk,),
        in_specs=[
            pl.BlockSpec((blk, OUT_CH), lambda i: (i, 0)),
            pl.BlockSpec((blk, OUT_CH), lambda i: (i, 0)),
            pl.BlockSpec((blk, OUT_CH), lambda i: (i, 0)),
            pl.BlockSpec((blk, 1), lambda i: (i, 0)),
            pl.BlockSpec((1, OUT_CH), lambda i: (0, 0)),
        ],
        out_specs=pl.BlockSpec((blk, OUT_CH), lambda i: (i, 0)),
        out_shape=jax.ShapeDtypeStruct((N_NODES, OUT_CH), jnp.float32),
    )(p0, p1, hs, dis2, b2)


def kernel(x, edge_index, edge_weight, W, b):
    nfill = DEG_CHUNKS * DCHUNK - N_EDGES
    src_f = jnp.concatenate(
        [edge_index[0].astype(jnp.int32), jnp.zeros((nfill,), jnp.int32)])
    dst_f = jnp.concatenate(
        [edge_index[1].astype(jnp.int32), jnp.zeros((nfill,), jnp.int32)])
    ew_f = jnp.concatenate(
        [edge_weight.astype(jnp.float32), jnp.zeros((nfill,), jnp.float32)])
    xp = jnp.pad(x.astype(jnp.float32), ((0, NPAD - N_NODES), (0, 0)))

    deg = _deg(dst_f.reshape(DEG_CHUNKS, DCHUNK), ew_f.reshape(DEG_CHUNKS, DCHUNK))
    hs, dis2 = _hs(deg.reshape(NC, NPAD).T, xp, W.astype(jnp.float32))
    parts = _msg(src_f.reshape(-1, CHUNK), dst_f.reshape(-1, CHUNK),
                 ew_f.reshape(-1, CHUNK), hs)
    return _combine(parts[0], parts[1], hs, dis2,
                    b.astype(jnp.float32).reshape(1, OUT_CH))
